# baseline (speedup 1.0000x reference)
import functools

import jax
import jax.numpy as jnp
from jax import lax
from jax.experimental import pallas as pl
from jax.experimental.pallas import tpu as pltpu
from jax.experimental.pallas import tpu_sc as plsc

N = 10000
E = 160000
F_IN = 256
H = 512
C = 40
EPS = 1e-5

NS = 16
NC = 2
ET = E // NS
BB = 128
NB = 80
ETP = NB * BB
NPAD = ETP
RPT = NPAD // NS
MB = 1000
GR = N // MB

_mesh = plsc.VectorSubcoreMesh(core_axis_name="c", subcore_axis_name="s")


def _sds(shape, dtype=jnp.float32):
    return jax.ShapeDtypeStruct(shape, dtype)


def _deg_body(dst_hbm, ones_hbm, zeros_hbm, deg_hbm, idx_v, ones_v, acc_sh):
    c = lax.axis_index("c")
    s = lax.axis_index("s")
    pltpu.sync_copy(dst_hbm.at[s], idx_v)
    pltpu.sync_copy(ones_hbm, ones_v)
    pltpu.sync_copy(zeros_hbm.at[pl.ds(s * RPT, RPT)],
                    acc_sh.at[pl.ds(s * RPT, RPT)])
    plsc.subcore_barrier()

    def body(j, carry):
        pltpu.sync_copy(ones_v, acc_sh.at[idx_v.at[2 * j + c]], add=True)
        return carry

    lax.fori_loop(0, NB // 2, body, 0)
    plsc.subcore_barrier()
    pltpu.sync_copy(acc_sh.at[pl.ds(s * RPT, RPT)],
                    deg_hbm.at[c, pl.ds(s * RPT, RPT)])


_deg_call = pl.kernel(
    _deg_body,
    out_type=_sds((NC, NPAD, 128)),
    mesh=_mesh,
    scratch_types=[
        pltpu.VMEM((NB, BB), jnp.int32),
        pltpu.VMEM((BB, 128), jnp.float32),
        pltpu.VMEM_SHARED((NPAD, 128), jnp.float32),
    ],
)


NBH = NB // 2


def _edge_pipeline(u_ref, srcv, dstv, acc_sh, bufs, gsem, ssem, nslots, row):
    @pl.loop(0, nslots)
    def _(i):
        pltpu.sync_copy(u_ref.at[srcv.at[row(i)]], bufs.at[0])
        pltpu.sync_copy(bufs.at[0], acc_sh.at[dstv.at[row(i)]], add=True)


def _agg_wide_body(u0, u1, u2, u3, src_hbm, dst_hbm, agg_hbm,
                   srcv, dstv, bufs, gsem, ssem, acc_sh):
    c = lax.axis_index("c")
    s = lax.axis_index("s")
    u_refs = (u0, u1, u2, u3)
    for chunk in range(4):
        u_ref = u_refs[chunk]

        @pl.when(c == chunk // 2)
        def _():
            pltpu.sync_copy(u_ref.at[pl.ds(s * RPT, RPT)],
                            acc_sh.at[pl.ds(s * RPT, RPT)])
            plsc.subcore_barrier()
            for p in range(2):
                pltpu.sync_copy(src_hbm.at[s, pl.ds(p * NBH, NBH)], srcv)
                pltpu.sync_copy(dst_hbm.at[s, pl.ds(p * NBH, NBH)], dstv)
                _edge_pipeline(u_ref, srcv, dstv, acc_sh, bufs, gsem, ssem,
                               NBH, lambda i: i)
            plsc.subcore_barrier()
            pltpu.sync_copy(acc_sh.at[pl.ds(s * RPT, RPT)],
                            agg_hbm.at[chunk, pl.ds(s * RPT, RPT)])
            plsc.subcore_barrier()


def _agg_scratch():
    return [
        pltpu.VMEM((NBH, BB), jnp.int32),
        pltpu.VMEM((NBH, BB), jnp.int32),
        pltpu.VMEM((2, BB, 128), jnp.float32),
        pltpu.SemaphoreType.DMA((2,)),
        pltpu.SemaphoreType.DMA((2,)),
        pltpu.VMEM_SHARED((NPAD, 128), jnp.float32),
    ]


_agg_wide_call = pl.kernel(
    _agg_wide_body,
    out_type=_sds((4, NPAD, 128)),
    mesh=_mesh,
    scratch_types=_agg_scratch(),
)


def _agg_narrow_body(u_hbm, src_hbm, dst_hbm, agg_hbm,
                     srcv, dstv, bufs, gsem, ssem, acc_sh):
    c = lax.axis_index("c")
    s = lax.axis_index("s")
    pltpu.sync_copy(u_hbm.at[pl.ds(s * RPT, RPT)],
                    acc_sh.at[pl.ds(s * RPT, RPT)])
    plsc.subcore_barrier()
    for p in range(2):
        pltpu.sync_copy(src_hbm.at[s, pl.ds(p * NBH, NBH)], srcv)
        pltpu.sync_copy(dst_hbm.at[s, pl.ds(p * NBH, NBH)], dstv)
        _edge_pipeline(u_hbm, srcv, dstv, acc_sh, bufs, gsem, ssem,
                       NBH // 2, lambda i: 2 * i + c)
    plsc.subcore_barrier()
    pltpu.sync_copy(acc_sh.at[pl.ds(s * RPT, RPT)],
                    agg_hbm.at[c, pl.ds(s * RPT, RPT)])


_agg_narrow_call = pl.kernel(
    _agg_narrow_body,
    out_type=_sds((NC, NPAD, 128)),
    mesh=_mesh,
    scratch_types=_agg_scratch(),
)


def _dinv_body(deg_ref, out_ref):
    d = deg_ref[...]
    out_ref[...] = lax.rsqrt(d[0] + d[1] + 1.0)


def _dinv_call(deg2):
    return pl.pallas_call(
        _dinv_body,
        out_shape=_sds((NPAD // 128, 128)),
    )(deg2)


def _mm1_body(x_ref, w_ref, dinv_ref, o0, o1, o2, o3):
    h = jnp.dot(x_ref[...], w_ref[...], preferred_element_type=jnp.float32)
    u = h * dinv_ref[...]
    for i, o in enumerate((o0, o1, o2, o3)):
        o[...] = u[:, i * 128:(i + 1) * 128]


def _mm1_call(x, w, dinv):
    return pl.pallas_call(
        _mm1_body,
        grid=(GR,),
        in_specs=[
            pl.BlockSpec((MB, F_IN), lambda i: (i, 0)),
            pl.BlockSpec((F_IN, H), lambda i: (0, 0)),
            pl.BlockSpec((MB, 1), lambda i: (i, 0)),
        ],
        out_specs=[pl.BlockSpec((MB, 128), lambda i: (i, 0))] * 4,
        out_shape=[_sds((NPAD, 128))] * 4,
    )(x, w, dinv)


def _stats_body(a0, a1, a2, a3, dinv_ref, b_ref, ps_ref, pq_ref):
    i = pl.program_id(0)
    y = jnp.concatenate([a0[...], a1[...], a2[...], a3[...]], axis=1)
    y = y * dinv_ref[...] + b_ref[...]

    @pl.when(i == 0)
    def _():
        ps_ref[...] = jnp.zeros_like(ps_ref)
        pq_ref[...] = jnp.zeros_like(pq_ref)

    ps_ref[...] += jnp.sum(y, axis=0, keepdims=True)
    pq_ref[...] += jnp.sum(y * y, axis=0, keepdims=True)


def _stats_call(chunks, dinv, b):
    return pl.pallas_call(
        _stats_body,
        grid=(GR,),
        in_specs=[pl.BlockSpec((MB, 128), lambda i: (i, 0))] * 4
        + [
            pl.BlockSpec((MB, 1), lambda i: (i, 0)),
            pl.BlockSpec((1, H), lambda i: (0, 0)),
        ],
        out_specs=[pl.BlockSpec((1, H), lambda i: (0, 0))] * 2,
        out_shape=[_sds((1, H))] * 2,
    )(*chunks, dinv, b)


def _bnmm_body(n_out, a0, a1, a2, a3, dinv_ref, b_ref, ps_ref, pq_ref,
               g_ref, be_ref, w_ref, *outs):
    mean = ps_ref[...] / N
    var = pq_ref[...] / N - mean * mean
    scale = g_ref[...] * lax.rsqrt(var + EPS)
    shift = be_ref[...] - mean * scale
    y = jnp.concatenate([a0[...], a1[...], a2[...], a3[...]], axis=1)
    y = (y * dinv_ref[...] + b_ref[...]) * scale + shift
    z = jnp.maximum(y, 0.0)
    u = jnp.dot(z, w_ref[...], preferred_element_type=jnp.float32)
    u = u * dinv_ref[...]
    if n_out == 1:
        outs[0][...] = u
    else:
        for i, o in enumerate(outs):
            o[...] = u[:, i * 128:(i + 1) * 128]


def _bnmm_call(chunks, dinv, b, ps, pq, g, be, w, n_out, wout):
    return pl.pallas_call(
        functools.partial(_bnmm_body, n_out),
        grid=(GR,),
        in_specs=[pl.BlockSpec((MB, 128), lambda i: (i, 0))] * 4
        + [
            pl.BlockSpec((MB, 1), lambda i: (i, 0)),
            pl.BlockSpec((1, H), lambda i: (0, 0)),
            pl.BlockSpec((1, H), lambda i: (0, 0)),
            pl.BlockSpec((1, H), lambda i: (0, 0)),
            pl.BlockSpec((1, H), lambda i: (0, 0)),
            pl.BlockSpec((1, H), lambda i: (0, 0)),
            pl.BlockSpec((H, wout * n_out), lambda i: (0, 0)),
        ],
        out_specs=[pl.BlockSpec((MB, wout), lambda i: (i, 0))] * n_out,
        out_shape=[_sds((NPAD, wout))] * n_out,
    )(*chunks, dinv, b, ps, pq, g, be, w)


def _final_body(s0, s1, u_ref, dinv_ref, b_ref, out_ref):
    r = (s0[...] + s1[...] - u_ref[...]) * dinv_ref[...]
    out_ref[...] = r[:, :C] + b_ref[...]


def _final_call(s0, s1, u, dinv, b):
    return pl.pallas_call(
        _final_body,
        grid=(GR,),
        in_specs=[
            pl.BlockSpec((MB, 128), lambda i: (i, 0)),
            pl.BlockSpec((MB, 128), lambda i: (i, 0)),
            pl.BlockSpec((MB, 128), lambda i: (i, 0)),
            pl.BlockSpec((MB, 1), lambda i: (i, 0)),
            pl.BlockSpec((1, C), lambda i: (0, 0)),
        ],
        out_specs=pl.BlockSpec((MB, C), lambda i: (i, 0)),
        out_shape=_sds((N, C)),
    )(s0, s1, u, dinv, b)


def kernel(x, edge_idx, W1, b1, g1, be1, W2, b2, g2, be2, W3, b3):
    src = edge_idx[0].reshape(NS, ET)
    dst = edge_idx[1].reshape(NS, ET)
    srcp = jnp.pad(src, ((0, 0), (0, ETP - ET))).reshape(NS, NB, BB)
    dstp = jnp.pad(dst, ((0, 0), (0, ETP - ET)),
                   constant_values=N).reshape(NS, NB, BB)
    ones = jnp.ones((BB, 128), jnp.float32)
    zeros = jnp.zeros((NPAD, 128), jnp.float32)

    deg = _deg_call(dstp, ones, zeros)
    deg2 = deg[:, :, 0].reshape(NC, NPAD // 128, 128)
    dinv = _dinv_call(deg2).reshape(NPAD)[:N].reshape(N, 1)

    b1r, g1r, be1r = b1.reshape(1, H), g1.reshape(1, H), be1.reshape(1, H)
    b2r, g2r, be2r = b2.reshape(1, H), g2.reshape(1, H), be2.reshape(1, H)
    b3r = b3.reshape(1, C)

    u1 = _mm1_call(x, W1, dinv)
    agg1 = _agg_wide_call(*u1, srcp, dstp)
    a1 = [agg1[i, :N] for i in range(4)]
    ps1, pq1 = _stats_call(a1, dinv, b1r)
    u2 = _bnmm_call(a1, dinv, b1r, ps1, pq1, g1r, be1r, W2, 4, 128)
    agg2 = _agg_wide_call(*u2, srcp, dstp)
    a2 = [agg2[i, :N] for i in range(4)]
    ps2, pq2 = _stats_call(a2, dinv, b2r)
    W3p = jnp.pad(W3, ((0, 0), (0, 128 - C)))
    (u3,) = _bnmm_call(a2, dinv, b2r, ps2, pq2, g2r, be2r, W3p, 1, 128)
    agg3 = _agg_narrow_call(u3, srcp, dstp)
    out = _final_call(agg3[0, :N], agg3[1, :N], u3[:N], dinv, b3r)
    return out


# speedup vs baseline: 1.0054x; 1.0054x over previous
import functools

import jax
import jax.numpy as jnp
from jax import lax
from jax.experimental import pallas as pl
from jax.experimental.pallas import tpu as pltpu
from jax.experimental.pallas import tpu_sc as plsc

N = 10000
E = 160000
F_IN = 256
H = 512
C = 40
EPS = 1e-5

NS = 16
NC = 2
ET = E // NS
BB = 128
NB = 80
ETP = NB * BB
NPAD = ETP
RPT = NPAD // NS
MB = 1000
GR = N // MB

_mesh = plsc.VectorSubcoreMesh(core_axis_name="c", subcore_axis_name="s")


def _sds(shape, dtype=jnp.float32):
    return jax.ShapeDtypeStruct(shape, dtype)


def _deg_body(dst_hbm, ones_hbm, zeros_hbm, deg_hbm, idx_v, ones_v, acc_sh):
    c = lax.axis_index("c")
    s = lax.axis_index("s")
    pltpu.sync_copy(dst_hbm.at[s], idx_v)
    pltpu.sync_copy(ones_hbm, ones_v)
    pltpu.sync_copy(zeros_hbm.at[pl.ds(s * RPT, RPT)],
                    acc_sh.at[pl.ds(s * RPT, RPT)])
    plsc.subcore_barrier()

    def body(j, carry):
        pltpu.sync_copy(ones_v, acc_sh.at[idx_v.at[2 * j + c]], add=True)
        return carry

    lax.fori_loop(0, NB // 2, body, 0)
    plsc.subcore_barrier()
    pltpu.sync_copy(acc_sh.at[pl.ds(s * RPT, RPT)],
                    deg_hbm.at[c, pl.ds(s * RPT, RPT)])


_deg_call = pl.kernel(
    _deg_body,
    out_type=_sds((NC, NPAD, 128)),
    mesh=_mesh,
    scratch_types=[
        pltpu.VMEM((NB, BB), jnp.int32),
        pltpu.VMEM((BB, 128), jnp.float32),
        pltpu.VMEM_SHARED((NPAD, 128), jnp.float32),
    ],
)


def _edge_pipeline(u_ref, srcv, dstv, acc_sh, rowbuf, nslots, row):
    @pl.loop(0, nslots)
    def _(i):
        pltpu.sync_copy(u_ref.at[srcv.at[row(i)]], rowbuf)
        pltpu.sync_copy(rowbuf, acc_sh.at[dstv.at[row(i)]], add=True)


def _agg_wide_body(u0, u1, u2, u3, src_hbm, dst_hbm, agg_hbm,
                   srcv, dstv, rowbuf, acc_sh):
    c = lax.axis_index("c")
    s = lax.axis_index("s")
    pltpu.sync_copy(src_hbm.at[s], srcv)
    pltpu.sync_copy(dst_hbm.at[s], dstv)
    u_refs = (u0, u1, u2, u3)
    for chunk in range(4):
        u_ref = u_refs[chunk]

        @pl.when(c == chunk // 2)
        def _():
            pltpu.sync_copy(u_ref.at[pl.ds(s * RPT, RPT)],
                            acc_sh.at[pl.ds(s * RPT, RPT)])
            plsc.subcore_barrier()
            _edge_pipeline(u_ref, srcv, dstv, acc_sh, rowbuf,
                           NB, lambda i: i)
            plsc.subcore_barrier()
            pltpu.sync_copy(acc_sh.at[pl.ds(s * RPT, RPT)],
                            agg_hbm.at[chunk, pl.ds(s * RPT, RPT)])
            plsc.subcore_barrier()


def _agg_scratch():
    return [
        pltpu.VMEM((NB, BB), jnp.int32),
        pltpu.VMEM((NB, BB), jnp.int32),
        pltpu.VMEM((BB, 128), jnp.float32),
        pltpu.VMEM_SHARED((NPAD, 128), jnp.float32),
    ]


_agg_wide_call = pl.kernel(
    _agg_wide_body,
    out_type=_sds((4, NPAD, 128)),
    mesh=_mesh,
    scratch_types=_agg_scratch(),
)


def _agg_narrow_body(u_hbm, src_hbm, dst_hbm, agg_hbm,
                     srcv, dstv, rowbuf, acc_sh):
    c = lax.axis_index("c")
    s = lax.axis_index("s")
    pltpu.sync_copy(src_hbm.at[s], srcv)
    pltpu.sync_copy(dst_hbm.at[s], dstv)
    pltpu.sync_copy(u_hbm.at[pl.ds(s * RPT, RPT)],
                    acc_sh.at[pl.ds(s * RPT, RPT)])
    plsc.subcore_barrier()
    _edge_pipeline(u_hbm, srcv, dstv, acc_sh, rowbuf,
                   NB // 2, lambda i: 2 * i + c)
    plsc.subcore_barrier()
    pltpu.sync_copy(acc_sh.at[pl.ds(s * RPT, RPT)],
                    agg_hbm.at[c, pl.ds(s * RPT, RPT)])


_agg_narrow_call = pl.kernel(
    _agg_narrow_body,
    out_type=_sds((NC, NPAD, 128)),
    mesh=_mesh,
    scratch_types=_agg_scratch(),
)


def _dinv_body(deg_ref, out_ref):
    d = deg_ref[...]
    out_ref[...] = lax.rsqrt(d[0] + d[1] + 1.0)


def _dinv_call(deg2):
    return pl.pallas_call(
        _dinv_body,
        out_shape=_sds((NPAD // 128, 128)),
    )(deg2)


def _mm1_body(x_ref, w_ref, dinv_ref, o0, o1, o2, o3):
    h = jnp.dot(x_ref[...], w_ref[...], preferred_element_type=jnp.float32)
    u = h * dinv_ref[...]
    for i, o in enumerate((o0, o1, o2, o3)):
        o[...] = u[:, i * 128:(i + 1) * 128]


def _mm1_call(x, w, dinv):
    return pl.pallas_call(
        _mm1_body,
        grid=(GR,),
        in_specs=[
            pl.BlockSpec((MB, F_IN), lambda i: (i, 0)),
            pl.BlockSpec((F_IN, H), lambda i: (0, 0)),
            pl.BlockSpec((MB, 1), lambda i: (i, 0)),
        ],
        out_specs=[pl.BlockSpec((MB, 128), lambda i: (i, 0))] * 4,
        out_shape=[_sds((NPAD, 128))] * 4,
    )(x, w, dinv)


def _stats_body(a0, a1, a2, a3, dinv_ref, b_ref, ps_ref, pq_ref):
    i = pl.program_id(0)
    y = jnp.concatenate([a0[...], a1[...], a2[...], a3[...]], axis=1)
    y = y * dinv_ref[...] + b_ref[...]

    @pl.when(i == 0)
    def _():
        ps_ref[...] = jnp.zeros_like(ps_ref)
        pq_ref[...] = jnp.zeros_like(pq_ref)

    ps_ref[...] += jnp.sum(y, axis=0, keepdims=True)
    pq_ref[...] += jnp.sum(y * y, axis=0, keepdims=True)


def _stats_call(chunks, dinv, b):
    return pl.pallas_call(
        _stats_body,
        grid=(GR,),
        in_specs=[pl.BlockSpec((MB, 128), lambda i: (i, 0))] * 4
        + [
            pl.BlockSpec((MB, 1), lambda i: (i, 0)),
            pl.BlockSpec((1, H), lambda i: (0, 0)),
        ],
        out_specs=[pl.BlockSpec((1, H), lambda i: (0, 0))] * 2,
        out_shape=[_sds((1, H))] * 2,
    )(*chunks, dinv, b)


def _bnmm_body(n_out, a0, a1, a2, a3, dinv_ref, b_ref, ps_ref, pq_ref,
               g_ref, be_ref, w_ref, *outs):
    mean = ps_ref[...] / N
    var = pq_ref[...] / N - mean * mean
    scale = g_ref[...] * lax.rsqrt(var + EPS)
    shift = be_ref[...] - mean * scale
    y = jnp.concatenate([a0[...], a1[...], a2[...], a3[...]], axis=1)
    y = (y * dinv_ref[...] + b_ref[...]) * scale + shift
    z = jnp.maximum(y, 0.0)
    u = jnp.dot(z, w_ref[...], preferred_element_type=jnp.float32)
    u = u * dinv_ref[...]
    if n_out == 1:
        outs[0][...] = u
    else:
        for i, o in enumerate(outs):
            o[...] = u[:, i * 128:(i + 1) * 128]


def _bnmm_call(chunks, dinv, b, ps, pq, g, be, w, n_out, wout):
    return pl.pallas_call(
        functools.partial(_bnmm_body, n_out),
        grid=(GR,),
        in_specs=[pl.BlockSpec((MB, 128), lambda i: (i, 0))] * 4
        + [
            pl.BlockSpec((MB, 1), lambda i: (i, 0)),
            pl.BlockSpec((1, H), lambda i: (0, 0)),
            pl.BlockSpec((1, H), lambda i: (0, 0)),
            pl.BlockSpec((1, H), lambda i: (0, 0)),
            pl.BlockSpec((1, H), lambda i: (0, 0)),
            pl.BlockSpec((1, H), lambda i: (0, 0)),
            pl.BlockSpec((H, wout * n_out), lambda i: (0, 0)),
        ],
        out_specs=[pl.BlockSpec((MB, wout), lambda i: (i, 0))] * n_out,
        out_shape=[_sds((NPAD, wout))] * n_out,
    )(*chunks, dinv, b, ps, pq, g, be, w)


def _final_body(s0, s1, u_ref, dinv_ref, b_ref, out_ref):
    r = (s0[...] + s1[...] - u_ref[...]) * dinv_ref[...]
    out_ref[...] = r[:, :C] + b_ref[...]


def _final_call(s0, s1, u, dinv, b):
    return pl.pallas_call(
        _final_body,
        grid=(GR,),
        in_specs=[
            pl.BlockSpec((MB, 128), lambda i: (i, 0)),
            pl.BlockSpec((MB, 128), lambda i: (i, 0)),
            pl.BlockSpec((MB, 128), lambda i: (i, 0)),
            pl.BlockSpec((MB, 1), lambda i: (i, 0)),
            pl.BlockSpec((1, C), lambda i: (0, 0)),
        ],
        out_specs=pl.BlockSpec((MB, C), lambda i: (i, 0)),
        out_shape=_sds((N, C)),
    )(s0, s1, u, dinv, b)


def kernel(x, edge_idx, W1, b1, g1, be1, W2, b2, g2, be2, W3, b3):
    src = edge_idx[0].reshape(NS, ET)
    dst = edge_idx[1].reshape(NS, ET)
    srcp = jnp.pad(src, ((0, 0), (0, ETP - ET))).reshape(NS, NB, BB)
    dstp = jnp.pad(dst, ((0, 0), (0, ETP - ET)),
                   constant_values=N).reshape(NS, NB, BB)
    ones = jnp.ones((BB, 128), jnp.float32)
    zeros = jnp.zeros((NPAD, 128), jnp.float32)

    deg = _deg_call(dstp, ones, zeros)
    deg2 = deg[:, :, 0].reshape(NC, NPAD // 128, 128)
    dinv = _dinv_call(deg2).reshape(NPAD)[:N].reshape(N, 1)

    b1r, g1r, be1r = b1.reshape(1, H), g1.reshape(1, H), be1.reshape(1, H)
    b2r, g2r, be2r = b2.reshape(1, H), g2.reshape(1, H), be2.reshape(1, H)
    b3r = b3.reshape(1, C)

    u1 = _mm1_call(x, W1, dinv)
    agg1 = _agg_wide_call(*u1, srcp, dstp)
    a1 = [agg1[i, :N] for i in range(4)]
    ps1, pq1 = _stats_call(a1, dinv, b1r)
    u2 = _bnmm_call(a1, dinv, b1r, ps1, pq1, g1r, be1r, W2, 4, 128)
    agg2 = _agg_wide_call(*u2, srcp, dstp)
    a2 = [agg2[i, :N] for i in range(4)]
    ps2, pq2 = _stats_call(a2, dinv, b2r)
    W3p = jnp.pad(W3, ((0, 0), (0, 128 - C)))
    (u3,) = _bnmm_call(a2, dinv, b2r, ps2, pq2, g2r, be2r, W3p, 1, 128)
    agg3 = _agg_narrow_call(u3, srcp, dstp)
    out = _final_call(agg3[0, :N], agg3[1, :N], u3[:N], dinv, b3r)
    return out


# speedup vs baseline: 1.6712x; 1.6623x over previous
import functools

import jax
import jax.numpy as jnp
from jax import lax
from jax.experimental import pallas as pl
from jax.experimental.pallas import tpu as pltpu
from jax.experimental.pallas import tpu_sc as plsc

N = 10000
E = 160000
F_IN = 256
H = 512
C = 40
EPS = 1e-5

NS = 16
NC = 2
ET = E // NS
BB = 128
NB = 80
ETP = NB * BB
NPAD = ETP
RPT = NPAD // NS
MB = 1000
GR = N // MB

_mesh = plsc.VectorSubcoreMesh(core_axis_name="c", subcore_axis_name="s")


def _sds(shape, dtype=jnp.float32):
    return jax.ShapeDtypeStruct(shape, dtype)


def _deg_body(dst_hbm, ones_hbm, zeros_hbm, deg_hbm, idx_v, ones_v, acc_sh):
    c = lax.axis_index("c")
    s = lax.axis_index("s")
    pltpu.sync_copy(dst_hbm.at[s], idx_v)
    pltpu.sync_copy(ones_hbm, ones_v)
    pltpu.sync_copy(zeros_hbm.at[pl.ds(s * RPT, RPT)],
                    acc_sh.at[pl.ds(s * RPT, RPT)])
    plsc.subcore_barrier()

    def body(j, carry):
        pltpu.sync_copy(ones_v, acc_sh.at[idx_v.at[2 * j + c]], add=True)
        return carry

    lax.fori_loop(0, NB // 2, body, 0)
    plsc.subcore_barrier()
    pltpu.sync_copy(acc_sh.at[pl.ds(s * RPT, RPT)],
                    deg_hbm.at[c, pl.ds(s * RPT, RPT)])


_deg_call = pl.kernel(
    _deg_body,
    out_type=_sds((NC, NPAD, 128)),
    mesh=_mesh,
    scratch_types=[
        pltpu.VMEM((NB, BB), jnp.int32),
        pltpu.VMEM((BB, 128), jnp.float32),
        pltpu.VMEM_SHARED((NPAD, 128), jnp.float32),
    ],
)


def _edge_pipeline(u_ref, srcv, dstv, acc_sh, rowbuf, nslots, row):
    @pl.loop(0, nslots)
    def _(i):
        pltpu.sync_copy(u_ref.at[srcv.at[row(i)]], rowbuf)
        pltpu.sync_copy(rowbuf, acc_sh.at[dstv.at[row(i)]], add=True)


def _agg_wide_body(u0, u1, u2, u3, src_hbm, dst_hbm, agg_hbm,
                   srcv, dstv, rowbuf, acc_sh):
    c = lax.axis_index("c")
    s = lax.axis_index("s")
    pltpu.sync_copy(src_hbm.at[s], srcv)
    pltpu.sync_copy(dst_hbm.at[s], dstv)
    u_refs = (u0, u1, u2, u3)
    for chunk in range(4):
        u_ref = u_refs[chunk]

        @pl.when(c == chunk // 2)
        def _():
            pltpu.sync_copy(u_ref.at[pl.ds(s * RPT, RPT)],
                            acc_sh.at[pl.ds(s * RPT, RPT)])
            plsc.subcore_barrier()
            _edge_pipeline(u_ref, srcv, dstv, acc_sh, rowbuf,
                           NB, lambda i: i)
            plsc.subcore_barrier()
            pltpu.sync_copy(acc_sh.at[pl.ds(s * RPT, RPT)],
                            agg_hbm.at[chunk, pl.ds(s * RPT, RPT)])
            plsc.subcore_barrier()


def _agg_scratch():
    return [
        pltpu.VMEM((NB, BB), jnp.int32),
        pltpu.VMEM((NB, BB), jnp.int32),
        pltpu.VMEM((BB, 128), jnp.float32),
        pltpu.VMEM_SHARED((NPAD, 128), jnp.float32),
    ]


_agg_wide_call = pl.kernel(
    _agg_wide_body,
    out_type=_sds((4, NPAD, 128)),
    mesh=_mesh,
    scratch_types=_agg_scratch(),
)


def _agg_narrow_body(u_hbm, src_hbm, dst_hbm, agg_hbm,
                     srcv, dstv, rowbuf, acc_sh):
    c = lax.axis_index("c")
    s = lax.axis_index("s")
    pltpu.sync_copy(src_hbm.at[s], srcv)
    pltpu.sync_copy(dst_hbm.at[s], dstv)
    pltpu.sync_copy(u_hbm.at[pl.ds(s * RPT, RPT)],
                    acc_sh.at[pl.ds(s * RPT, RPT)])
    plsc.subcore_barrier()
    _edge_pipeline(u_hbm, srcv, dstv, acc_sh, rowbuf,
                   NB // 2, lambda i: 2 * i + c)
    plsc.subcore_barrier()
    pltpu.sync_copy(acc_sh.at[pl.ds(s * RPT, RPT)],
                    agg_hbm.at[c, pl.ds(s * RPT, RPT)])


_agg_narrow_call = pl.kernel(
    _agg_narrow_body,
    out_type=_sds((NC, NPAD, 128)),
    mesh=_mesh,
    scratch_types=_agg_scratch(),
)


def _dinv_body(deg_ref, out_ref):
    d = deg_ref[...]
    out_ref[...] = lax.rsqrt(d[0] + d[1] + 1.0)


def _dinv_call(deg2):
    return pl.pallas_call(
        _dinv_body,
        out_shape=_sds((NPAD // 128, 128)),
    )(deg2)


def _mm1_body(x_ref, w_ref, dinv_ref, o0, o1, o2, o3):
    h = jnp.dot(x_ref[...], w_ref[...], preferred_element_type=jnp.float32)
    u = h * dinv_ref[...]
    for i, o in enumerate((o0, o1, o2, o3)):
        o[...] = u[:, i * 128:(i + 1) * 128]


def _mm1_call(x, w, dinv):
    return pl.pallas_call(
        _mm1_body,
        grid=(GR,),
        in_specs=[
            pl.BlockSpec((MB, F_IN), lambda i: (i, 0)),
            pl.BlockSpec((F_IN, H), lambda i: (0, 0)),
            pl.BlockSpec((MB, 1), lambda i: (i, 0)),
        ],
        out_specs=[pl.BlockSpec((MB, 128), lambda i: (i, 0))] * 4,
        out_shape=[_sds((NPAD, 128))] * 4,
    )(x, w, dinv)


def _stats_body(a0, a1, a2, a3, dinv_ref, b_ref, ps_ref, pq_ref):
    i = pl.program_id(0)
    y = jnp.concatenate([a0[...], a1[...], a2[...], a3[...]], axis=1)
    y = y * dinv_ref[...] + b_ref[...]

    @pl.when(i == 0)
    def _():
        ps_ref[...] = jnp.zeros_like(ps_ref)
        pq_ref[...] = jnp.zeros_like(pq_ref)

    ps_ref[...] += jnp.sum(y, axis=0, keepdims=True)
    pq_ref[...] += jnp.sum(y * y, axis=0, keepdims=True)


def _stats_call(chunks, dinv, b):
    return pl.pallas_call(
        _stats_body,
        grid=(GR,),
        in_specs=[pl.BlockSpec((MB, 128), lambda i: (i, 0))] * 4
        + [
            pl.BlockSpec((MB, 1), lambda i: (i, 0)),
            pl.BlockSpec((1, H), lambda i: (0, 0)),
        ],
        out_specs=[pl.BlockSpec((1, H), lambda i: (0, 0))] * 2,
        out_shape=[_sds((1, H))] * 2,
    )(*chunks, dinv, b)


def _bnmm_body(n_out, a0, a1, a2, a3, dinv_ref, b_ref, ps_ref, pq_ref,
               g_ref, be_ref, w_ref, *outs):
    mean = ps_ref[...] / N
    var = pq_ref[...] / N - mean * mean
    scale = g_ref[...] * lax.rsqrt(var + EPS)
    shift = be_ref[...] - mean * scale
    y = jnp.concatenate([a0[...], a1[...], a2[...], a3[...]], axis=1)
    y = (y * dinv_ref[...] + b_ref[...]) * scale + shift
    z = jnp.maximum(y, 0.0)
    u = jnp.dot(z, w_ref[...], preferred_element_type=jnp.float32)
    u = u * dinv_ref[...]
    if n_out == 1:
        outs[0][...] = u
    else:
        for i, o in enumerate(outs):
            o[...] = u[:, i * 128:(i + 1) * 128]


def _bnmm_call(chunks, dinv, b, ps, pq, g, be, w, n_out, wout):
    return pl.pallas_call(
        functools.partial(_bnmm_body, n_out),
        grid=(GR,),
        in_specs=[pl.BlockSpec((MB, 128), lambda i: (i, 0))] * 4
        + [
            pl.BlockSpec((MB, 1), lambda i: (i, 0)),
            pl.BlockSpec((1, H), lambda i: (0, 0)),
            pl.BlockSpec((1, H), lambda i: (0, 0)),
            pl.BlockSpec((1, H), lambda i: (0, 0)),
            pl.BlockSpec((1, H), lambda i: (0, 0)),
            pl.BlockSpec((1, H), lambda i: (0, 0)),
            pl.BlockSpec((H, wout * n_out), lambda i: (0, 0)),
        ],
        out_specs=[pl.BlockSpec((MB, wout), lambda i: (i, 0))] * n_out,
        out_shape=[_sds((NPAD, wout))] * n_out,
    )(*chunks, dinv, b, ps, pq, g, be, w)


def _final_body(s0, s1, u_ref, dinv_ref, b_ref, out_ref):
    r = (s0[...] + s1[...] - u_ref[...]) * dinv_ref[...]
    out_ref[...] = r[:, :C] + b_ref[...]


def _final_call(s0, s1, u, dinv, b):
    return pl.pallas_call(
        _final_body,
        grid=(GR,),
        in_specs=[
            pl.BlockSpec((MB, 128), lambda i: (i, 0)),
            pl.BlockSpec((MB, 128), lambda i: (i, 0)),
            pl.BlockSpec((MB, 128), lambda i: (i, 0)),
            pl.BlockSpec((MB, 1), lambda i: (i, 0)),
            pl.BlockSpec((1, C), lambda i: (0, 0)),
        ],
        out_specs=pl.BlockSpec((MB, C), lambda i: (i, 0)),
        out_shape=_sds((N, C)),
    )(s0, s1, u, dinv, b)


def kernel(x, edge_idx, W1, b1, g1, be1, W2, b2, g2, be2, W3, b3):
    src = edge_idx[0].reshape(NS, ET)
    dst = edge_idx[1].reshape(NS, ET)
    spp = (NPAD - N) // NS
    pad = (N + jnp.arange(NS, dtype=jnp.int32)[:, None] * spp
           + jnp.arange(ETP - ET, dtype=jnp.int32)[None, :] % spp)
    srcp = jnp.concatenate([src, pad], axis=1).reshape(NS, NB, BB)
    dstp = jnp.concatenate([dst, pad], axis=1).reshape(NS, NB, BB)
    ones = jnp.ones((BB, 128), jnp.float32)
    zeros = jnp.zeros((NPAD, 128), jnp.float32)

    deg = _deg_call(dstp, ones, zeros)
    deg2 = deg[:, :, 0].reshape(NC, NPAD // 128, 128)
    dinv = _dinv_call(deg2).reshape(NPAD)[:N].reshape(N, 1)

    b1r, g1r, be1r = b1.reshape(1, H), g1.reshape(1, H), be1.reshape(1, H)
    b2r, g2r, be2r = b2.reshape(1, H), g2.reshape(1, H), be2.reshape(1, H)
    b3r = b3.reshape(1, C)

    u1 = _mm1_call(x, W1, dinv)
    agg1 = _agg_wide_call(*u1, srcp, dstp)
    a1 = [agg1[i, :N] for i in range(4)]
    ps1, pq1 = _stats_call(a1, dinv, b1r)
    u2 = _bnmm_call(a1, dinv, b1r, ps1, pq1, g1r, be1r, W2, 4, 128)
    agg2 = _agg_wide_call(*u2, srcp, dstp)
    a2 = [agg2[i, :N] for i in range(4)]
    ps2, pq2 = _stats_call(a2, dinv, b2r)
    W3p = jnp.pad(W3, ((0, 0), (0, 128 - C)))
    (u3,) = _bnmm_call(a2, dinv, b2r, ps2, pq2, g2r, be2r, W3p, 1, 128)
    agg3 = _agg_narrow_call(u3, srcp, dstp)
    out = _final_call(agg3[0, :N], agg3[1, :N], u3[:N], dinv, b3r)
    return out


# speedup vs baseline: 2.0652x; 1.2358x over previous
import functools

import jax
import jax.numpy as jnp
from jax import lax
from jax.experimental import pallas as pl
from jax.experimental.pallas import tpu as pltpu
from jax.experimental.pallas import tpu_sc as plsc

N = 10000
E = 160000
F_IN = 256
H = 512
C = 40
EPS = 1e-5

NS = 16
NC = 2
ET = E // NS
BB = 128
NB = 80
ETP = NB * BB
NPAD = ETP
RPT = NPAD // NS
MB = 1000
GR = N // MB

_mesh = plsc.VectorSubcoreMesh(core_axis_name="c", subcore_axis_name="s")


def _sds(shape, dtype=jnp.float32):
    return jax.ShapeDtypeStruct(shape, dtype)


def _deg_body(dst_hbm, ones_hbm, zeros_hbm, deg_hbm, idx_v, ones_v, acc_sh):
    c = lax.axis_index("c")
    s = lax.axis_index("s")
    pltpu.sync_copy(dst_hbm.at[s], idx_v)
    pltpu.sync_copy(ones_hbm, ones_v)
    pltpu.sync_copy(zeros_hbm.at[pl.ds(s * RPT, RPT)],
                    acc_sh.at[pl.ds(s * RPT, RPT)])
    plsc.subcore_barrier()

    def body(j, carry):
        pltpu.sync_copy(ones_v, acc_sh.at[idx_v.at[2 * j + c]], add=True)
        return carry

    lax.fori_loop(0, NB // 2, body, 0)
    plsc.subcore_barrier()
    pltpu.sync_copy(acc_sh.at[pl.ds(s * RPT, RPT)],
                    deg_hbm.at[c, pl.ds(s * RPT, RPT)])


_deg_call = pl.kernel(
    _deg_body,
    out_type=_sds((NC, NPAD, 128)),
    mesh=_mesh,
    scratch_types=[
        pltpu.VMEM((NB, BB), jnp.int32),
        pltpu.VMEM((BB, 128), jnp.float32),
        pltpu.VMEM_SHARED((NPAD, 128), jnp.float32),
    ],
)


NBH = NB // 2


def _edge_pipeline(u_ref, srcv, dstv, acc_sh, bufs, gsem, ssem, nslots, row):
    def gstart(i, b):
        pltpu.make_async_copy(u_ref.at[srcv.at[row(i)]], bufs.at[b],
                              gsem.at[b]).start()

    def gwait(b):
        pltpu.make_async_copy(u_ref.at[srcv.at[0]], bufs.at[b],
                              gsem.at[b]).wait()

    def sstart(i, b):
        pltpu.make_async_copy(bufs.at[b], acc_sh.at[dstv.at[row(i)]],
                              ssem.at[b]).start(add=True)

    def swait(b):
        pltpu.make_async_copy(bufs.at[0], acc_sh.at[dstv.at[0]],
                              ssem.at[b]).wait()

    gstart(0, 0)

    @pl.loop(0, nslots, step=2)
    def _(j):
        for b in range(2):
            i = j + b
            gwait(b)

            @pl.when(i >= 1)
            def _():
                swait(1 - b)

            @pl.when(i + 1 < nslots)
            def _():
                gstart(i + 1, 1 - b)

            sstart(i, b)

    swait((nslots - 1) % 2)


def _agg_wide_body(u0, u1, u2, u3, src_hbm, dst_hbm, agg_hbm,
                   srcv, dstv, bufs, gsem, ssem, acc_sh):
    c = lax.axis_index("c")
    s = lax.axis_index("s")
    u_refs = (u0, u1, u2, u3)
    for chunk in range(4):
        u_ref = u_refs[chunk]

        @pl.when(c == chunk // 2)
        def _():
            pltpu.sync_copy(u_ref.at[pl.ds(s * RPT, RPT)],
                            acc_sh.at[pl.ds(s * RPT, RPT)])
            plsc.subcore_barrier()
            for p in range(2):
                pltpu.sync_copy(src_hbm.at[s, pl.ds(p * NBH, NBH)], srcv)
                pltpu.sync_copy(dst_hbm.at[s, pl.ds(p * NBH, NBH)], dstv)
                _edge_pipeline(u_ref, srcv, dstv, acc_sh, bufs, gsem, ssem,
                               NBH, lambda i: i)
            plsc.subcore_barrier()
            pltpu.sync_copy(acc_sh.at[pl.ds(s * RPT, RPT)],
                            agg_hbm.at[chunk, pl.ds(s * RPT, RPT)])
            plsc.subcore_barrier()


def _agg_scratch():
    return [
        pltpu.VMEM((NBH, BB), jnp.int32),
        pltpu.VMEM((NBH, BB), jnp.int32),
        pltpu.VMEM((2, BB, 128), jnp.float32),
        pltpu.SemaphoreType.DMA((2,)),
        pltpu.SemaphoreType.DMA((2,)),
        pltpu.VMEM_SHARED((NPAD, 128), jnp.float32),
    ]


_agg_wide_call = pl.kernel(
    _agg_wide_body,
    out_type=_sds((4, NPAD, 128)),
    mesh=_mesh,
    scratch_types=_agg_scratch(),
)


def _agg_narrow_body(u_hbm, src_hbm, dst_hbm, agg_hbm,
                     srcv, dstv, bufs, gsem, ssem, acc_sh):
    c = lax.axis_index("c")
    s = lax.axis_index("s")
    pltpu.sync_copy(u_hbm.at[pl.ds(s * RPT, RPT)],
                    acc_sh.at[pl.ds(s * RPT, RPT)])
    plsc.subcore_barrier()
    for p in range(2):
        pltpu.sync_copy(src_hbm.at[s, pl.ds(p * NBH, NBH)], srcv)
        pltpu.sync_copy(dst_hbm.at[s, pl.ds(p * NBH, NBH)], dstv)
        _edge_pipeline(u_hbm, srcv, dstv, acc_sh, bufs, gsem, ssem,
                       NBH // 2, lambda i: 2 * i + c)
    plsc.subcore_barrier()
    pltpu.sync_copy(acc_sh.at[pl.ds(s * RPT, RPT)],
                    agg_hbm.at[c, pl.ds(s * RPT, RPT)])


_agg_narrow_call = pl.kernel(
    _agg_narrow_body,
    out_type=_sds((NC, NPAD, 128)),
    mesh=_mesh,
    scratch_types=_agg_scratch(),
)


def _dinv_body(deg_ref, out_ref):
    d = deg_ref[...]
    out_ref[...] = lax.rsqrt(d[0] + d[1] + 1.0)


def _dinv_call(deg2):
    return pl.pallas_call(
        _dinv_body,
        out_shape=_sds((NPAD // 128, 128)),
    )(deg2)


def _mm1_body(x_ref, w_ref, dinv_ref, o0, o1, o2, o3):
    h = jnp.dot(x_ref[...], w_ref[...], preferred_element_type=jnp.float32)
    u = h * dinv_ref[...]
    for i, o in enumerate((o0, o1, o2, o3)):
        o[...] = u[:, i * 128:(i + 1) * 128]


def _mm1_call(x, w, dinv):
    return pl.pallas_call(
        _mm1_body,
        grid=(GR,),
        in_specs=[
            pl.BlockSpec((MB, F_IN), lambda i: (i, 0)),
            pl.BlockSpec((F_IN, H), lambda i: (0, 0)),
            pl.BlockSpec((MB, 1), lambda i: (i, 0)),
        ],
        out_specs=[pl.BlockSpec((MB, 128), lambda i: (i, 0))] * 4,
        out_shape=[_sds((NPAD, 128))] * 4,
    )(x, w, dinv)


def _stats_body(a0, a1, a2, a3, dinv_ref, b_ref, ps_ref, pq_ref):
    i = pl.program_id(0)
    y = jnp.concatenate([a0[...], a1[...], a2[...], a3[...]], axis=1)
    y = y * dinv_ref[...] + b_ref[...]

    @pl.when(i == 0)
    def _():
        ps_ref[...] = jnp.zeros_like(ps_ref)
        pq_ref[...] = jnp.zeros_like(pq_ref)

    ps_ref[...] += jnp.sum(y, axis=0, keepdims=True)
    pq_ref[...] += jnp.sum(y * y, axis=0, keepdims=True)


def _stats_call(chunks, dinv, b):
    return pl.pallas_call(
        _stats_body,
        grid=(GR,),
        in_specs=[pl.BlockSpec((MB, 128), lambda i: (i, 0))] * 4
        + [
            pl.BlockSpec((MB, 1), lambda i: (i, 0)),
            pl.BlockSpec((1, H), lambda i: (0, 0)),
        ],
        out_specs=[pl.BlockSpec((1, H), lambda i: (0, 0))] * 2,
        out_shape=[_sds((1, H))] * 2,
    )(*chunks, dinv, b)


def _bnmm_body(n_out, a0, a1, a2, a3, dinv_ref, b_ref, ps_ref, pq_ref,
               g_ref, be_ref, w_ref, *outs):
    mean = ps_ref[...] / N
    var = pq_ref[...] / N - mean * mean
    scale = g_ref[...] * lax.rsqrt(var + EPS)
    shift = be_ref[...] - mean * scale
    y = jnp.concatenate([a0[...], a1[...], a2[...], a3[...]], axis=1)
    y = (y * dinv_ref[...] + b_ref[...]) * scale + shift
    z = jnp.maximum(y, 0.0)
    u = jnp.dot(z, w_ref[...], preferred_element_type=jnp.float32)
    u = u * dinv_ref[...]
    if n_out == 1:
        outs[0][...] = u
    else:
        for i, o in enumerate(outs):
            o[...] = u[:, i * 128:(i + 1) * 128]


def _bnmm_call(chunks, dinv, b, ps, pq, g, be, w, n_out, wout):
    return pl.pallas_call(
        functools.partial(_bnmm_body, n_out),
        grid=(GR,),
        in_specs=[pl.BlockSpec((MB, 128), lambda i: (i, 0))] * 4
        + [
            pl.BlockSpec((MB, 1), lambda i: (i, 0)),
            pl.BlockSpec((1, H), lambda i: (0, 0)),
            pl.BlockSpec((1, H), lambda i: (0, 0)),
            pl.BlockSpec((1, H), lambda i: (0, 0)),
            pl.BlockSpec((1, H), lambda i: (0, 0)),
            pl.BlockSpec((1, H), lambda i: (0, 0)),
            pl.BlockSpec((H, wout * n_out), lambda i: (0, 0)),
        ],
        out_specs=[pl.BlockSpec((MB, wout), lambda i: (i, 0))] * n_out,
        out_shape=[_sds((NPAD, wout))] * n_out,
    )(*chunks, dinv, b, ps, pq, g, be, w)


def _final_body(s0, s1, u_ref, dinv_ref, b_ref, out_ref):
    r = (s0[...] + s1[...] - u_ref[...]) * dinv_ref[...]
    out_ref[...] = r[:, :C] + b_ref[...]


def _final_call(s0, s1, u, dinv, b):
    return pl.pallas_call(
        _final_body,
        grid=(GR,),
        in_specs=[
            pl.BlockSpec((MB, 128), lambda i: (i, 0)),
            pl.BlockSpec((MB, 128), lambda i: (i, 0)),
            pl.BlockSpec((MB, 128), lambda i: (i, 0)),
            pl.BlockSpec((MB, 1), lambda i: (i, 0)),
            pl.BlockSpec((1, C), lambda i: (0, 0)),
        ],
        out_specs=pl.BlockSpec((MB, C), lambda i: (i, 0)),
        out_shape=_sds((N, C)),
    )(s0, s1, u, dinv, b)


def kernel(x, edge_idx, W1, b1, g1, be1, W2, b2, g2, be2, W3, b3):
    src = edge_idx[0].reshape(NS, ET)
    dst = edge_idx[1].reshape(NS, ET)
    spp = (NPAD - N) // NS
    pad = (N + jnp.arange(NS, dtype=jnp.int32)[:, None] * spp
           + jnp.arange(ETP - ET, dtype=jnp.int32)[None, :] % spp)
    srcp = jnp.concatenate([src, pad], axis=1).reshape(NS, NB, BB)
    dstp = jnp.concatenate([dst, pad], axis=1).reshape(NS, NB, BB)
    ones = jnp.ones((BB, 128), jnp.float32)
    zeros = jnp.zeros((NPAD, 128), jnp.float32)

    deg = _deg_call(dstp, ones, zeros)
    deg2 = deg[:, :, 0].reshape(NC, NPAD // 128, 128)
    dinv = _dinv_call(deg2).reshape(NPAD)[:N].reshape(N, 1)

    b1r, g1r, be1r = b1.reshape(1, H), g1.reshape(1, H), be1.reshape(1, H)
    b2r, g2r, be2r = b2.reshape(1, H), g2.reshape(1, H), be2.reshape(1, H)
    b3r = b3.reshape(1, C)

    u1 = _mm1_call(x, W1, dinv)
    agg1 = _agg_wide_call(*u1, srcp, dstp)
    a1 = [agg1[i, :N] for i in range(4)]
    ps1, pq1 = _stats_call(a1, dinv, b1r)
    u2 = _bnmm_call(a1, dinv, b1r, ps1, pq1, g1r, be1r, W2, 4, 128)
    agg2 = _agg_wide_call(*u2, srcp, dstp)
    a2 = [agg2[i, :N] for i in range(4)]
    ps2, pq2 = _stats_call(a2, dinv, b2r)
    W3p = jnp.pad(W3, ((0, 0), (0, 128 - C)))
    (u3,) = _bnmm_call(a2, dinv, b2r, ps2, pq2, g2r, be2r, W3p, 1, 128)
    agg3 = _agg_narrow_call(u3, srcp, dstp)
    out = _final_call(agg3[0, :N], agg3[1, :N], u3[:N], dinv, b3r)
    return out


# speedup vs baseline: 2.1492x; 1.0407x over previous
import functools

import jax
import jax.numpy as jnp
from jax import lax
from jax.experimental import pallas as pl
from jax.experimental.pallas import tpu as pltpu
from jax.experimental.pallas import tpu_sc as plsc

N = 10000
E = 160000
F_IN = 256
H = 512
C = 40
EPS = 1e-5

NS = 16
NC = 2
ET = E // NS
BB = 128
NB = 80
ETP = NB * BB
NPAD = ETP
RPT = NPAD // NS
MB = 1000
GR = N // MB

_mesh = plsc.VectorSubcoreMesh(core_axis_name="c", subcore_axis_name="s")


def _sds(shape, dtype=jnp.float32):
    return jax.ShapeDtypeStruct(shape, dtype)


def _deg_body(dst_hbm, ones_hbm, zeros_hbm, deg_hbm, idx_v, ones_v, acc_sh):
    c = lax.axis_index("c")
    s = lax.axis_index("s")
    pltpu.sync_copy(dst_hbm.at[s], idx_v)
    pltpu.sync_copy(ones_hbm, ones_v)
    pltpu.sync_copy(zeros_hbm.at[pl.ds(s * RPT, RPT)],
                    acc_sh.at[pl.ds(s * RPT, RPT)])
    plsc.subcore_barrier()

    def body(j, carry):
        pltpu.sync_copy(ones_v, acc_sh.at[idx_v.at[2 * j + c]], add=True)
        return carry

    lax.fori_loop(0, NB // 2, body, 0)
    plsc.subcore_barrier()
    pltpu.sync_copy(acc_sh.at[pl.ds(s * RPT, RPT)],
                    deg_hbm.at[c, pl.ds(s * RPT, RPT)])


_deg_call = pl.kernel(
    _deg_body,
    out_type=_sds((NC, NPAD, 128)),
    mesh=_mesh,
    scratch_types=[
        pltpu.VMEM((NB, BB), jnp.int32),
        pltpu.VMEM((BB, 128), jnp.float32),
        pltpu.VMEM_SHARED((NPAD, 128), jnp.float32),
    ],
)


NBH = NB // 2


def _edge_pipeline(u_ref, srcv, dstv, acc_sh, bufs, gsem, ssem, nslots, row):
    def gstart(i, b):
        pltpu.make_async_copy(u_ref.at[srcv.at[row(i)]], bufs.at[b],
                              gsem.at[b]).start()

    def gwait(b):
        pltpu.make_async_copy(u_ref.at[srcv.at[0]], bufs.at[b],
                              gsem.at[b]).wait()

    def sstart(i, b):
        pltpu.make_async_copy(bufs.at[b], acc_sh.at[dstv.at[row(i)]],
                              ssem.at[b]).start(add=True)

    def swait(b):
        pltpu.make_async_copy(bufs.at[0], acc_sh.at[dstv.at[0]],
                              ssem.at[b]).wait()

    gstart(0, 0)

    @pl.loop(0, nslots, step=2)
    def _(j):
        for b in range(2):
            i = j + b
            gwait(b)

            @pl.when(i >= 1)
            def _():
                swait(1 - b)

            @pl.when(i + 1 < nslots)
            def _():
                gstart(i + 1, 1 - b)

            sstart(i, b)

    swait((nslots - 1) % 2)


def _agg_wide_body(u0, u1, u2, u3, src_hbm, dst_hbm, agg_hbm,
                   srcv, dstv, bufs, gsem, ssem, acc_sh):
    c = lax.axis_index("c")
    s = lax.axis_index("s")
    u_refs = (u0, u1, u2, u3)
    for chunk in range(4):
        u_ref = u_refs[chunk]

        @pl.when(c == chunk // 2)
        def _():
            pltpu.sync_copy(u_ref.at[pl.ds(s * RPT, RPT)],
                            acc_sh.at[pl.ds(s * RPT, RPT)])
            plsc.subcore_barrier()
            for p in range(2):
                pltpu.sync_copy(src_hbm.at[s, pl.ds(p * NBH, NBH)], srcv)
                pltpu.sync_copy(dst_hbm.at[s, pl.ds(p * NBH, NBH)], dstv)
                _edge_pipeline(u_ref, srcv, dstv, acc_sh, bufs, gsem, ssem,
                               NBH, lambda i: i)
            plsc.subcore_barrier()
            pltpu.sync_copy(acc_sh.at[pl.ds(s * RPT, RPT)],
                            agg_hbm.at[chunk, pl.ds(s * RPT, RPT)])
            plsc.subcore_barrier()


def _agg_scratch():
    return [
        pltpu.VMEM((NBH, BB), jnp.int32),
        pltpu.VMEM((NBH, BB), jnp.int32),
        pltpu.VMEM((2, BB, 128), jnp.float32),
        pltpu.SemaphoreType.DMA((2,)),
        pltpu.SemaphoreType.DMA((2,)),
        pltpu.VMEM_SHARED((NPAD, 128), jnp.float32),
    ]


_agg_wide_call = pl.kernel(
    _agg_wide_body,
    out_type=_sds((4, NPAD, 128)),
    mesh=_mesh,
    scratch_types=_agg_scratch(),
)


def _agg_narrow_body(u_hbm, src_hbm, dst_hbm, agg_hbm,
                     srcv, dstv, bufs, gsem, ssem, acc_sh):
    c = lax.axis_index("c")
    s = lax.axis_index("s")
    pltpu.sync_copy(u_hbm.at[pl.ds(s * RPT, RPT)],
                    acc_sh.at[pl.ds(s * RPT, RPT)])
    plsc.subcore_barrier()
    for p in range(2):
        pltpu.sync_copy(src_hbm.at[s, pl.ds(p * NBH, NBH)], srcv)
        pltpu.sync_copy(dst_hbm.at[s, pl.ds(p * NBH, NBH)], dstv)
        _edge_pipeline(u_hbm, srcv, dstv, acc_sh, bufs, gsem, ssem,
                       NBH // 2, lambda i: 2 * i + c)
    plsc.subcore_barrier()
    pltpu.sync_copy(acc_sh.at[pl.ds(s * RPT, RPT)],
                    agg_hbm.at[c, pl.ds(s * RPT, RPT)])


_agg_narrow_call = pl.kernel(
    _agg_narrow_body,
    out_type=_sds((NC, NPAD, 128)),
    mesh=_mesh,
    scratch_types=_agg_scratch(),
)


def _dinv_body(deg_ref, out_ref):
    d = deg_ref[...]
    out_ref[...] = lax.rsqrt(d[0] + d[1] + 1.0)


def _dinv_call(deg2):
    return pl.pallas_call(
        _dinv_body,
        out_shape=_sds((NPAD // 128, 128)),
    )(deg2)


def _mm1_body(x_ref, w_ref, dinv_ref, o0, o1, o2, o3):
    h = jnp.dot(x_ref[...], w_ref[...], preferred_element_type=jnp.float32)
    u = h * dinv_ref[...]
    for i, o in enumerate((o0, o1, o2, o3)):
        o[...] = u[:, i * 128:(i + 1) * 128]


def _mm1_call(x, w, dinv):
    return pl.pallas_call(
        _mm1_body,
        grid=(GR,),
        in_specs=[
            pl.BlockSpec((MB, F_IN), lambda i: (i, 0)),
            pl.BlockSpec((F_IN, H), lambda i: (0, 0)),
            pl.BlockSpec((MB, 1), lambda i: (i, 0)),
        ],
        out_specs=[pl.BlockSpec((MB, 128), lambda i: (i, 0))] * 4,
        out_shape=[_sds((NPAD, 128))] * 4,
    )(x, w, dinv)


def _stats_body(a0, a1, a2, a3, dinv_ref, b_ref, ps_ref, pq_ref):
    i = pl.program_id(0)
    y = jnp.concatenate([a0[0], a1[0], a2[0], a3[0]], axis=1)
    y = y * dinv_ref[...] + b_ref[...]

    @pl.when(i == 0)
    def _():
        ps_ref[...] = jnp.zeros_like(ps_ref)
        pq_ref[...] = jnp.zeros_like(pq_ref)

    ps_ref[...] += jnp.sum(y, axis=0, keepdims=True)
    pq_ref[...] += jnp.sum(y * y, axis=0, keepdims=True)


def _stats_call(agg, dinv, b):
    return pl.pallas_call(
        _stats_body,
        grid=(GR,),
        in_specs=[pl.BlockSpec((1, MB, 128), lambda i, ch=ch: (ch, i, 0))
                  for ch in range(4)]
        + [
            pl.BlockSpec((MB, 1), lambda i: (i, 0)),
            pl.BlockSpec((1, H), lambda i: (0, 0)),
        ],
        out_specs=[pl.BlockSpec((1, H), lambda i: (0, 0))] * 2,
        out_shape=[_sds((1, H))] * 2,
    )(agg, agg, agg, agg, dinv, b)


def _bnmm_body(n_out, a0, a1, a2, a3, dinv_ref, b_ref, ps_ref, pq_ref,
               g_ref, be_ref, w_ref, *outs):
    mean = ps_ref[...] / N
    var = pq_ref[...] / N - mean * mean
    scale = g_ref[...] * lax.rsqrt(var + EPS)
    shift = be_ref[...] - mean * scale
    y = jnp.concatenate([a0[0], a1[0], a2[0], a3[0]], axis=1)
    y = (y * dinv_ref[...] + b_ref[...]) * scale + shift
    z = jnp.maximum(y, 0.0)
    u = jnp.dot(z, w_ref[...], preferred_element_type=jnp.float32)
    u = u * dinv_ref[...]
    if n_out == 1:
        outs[0][...] = u
    else:
        for i, o in enumerate(outs):
            o[...] = u[:, i * 128:(i + 1) * 128]


def _bnmm_call(agg, dinv, b, ps, pq, g, be, w, n_out, wout):
    return pl.pallas_call(
        functools.partial(_bnmm_body, n_out),
        grid=(GR,),
        in_specs=[pl.BlockSpec((1, MB, 128), lambda i, ch=ch: (ch, i, 0))
                  for ch in range(4)]
        + [
            pl.BlockSpec((MB, 1), lambda i: (i, 0)),
            pl.BlockSpec((1, H), lambda i: (0, 0)),
            pl.BlockSpec((1, H), lambda i: (0, 0)),
            pl.BlockSpec((1, H), lambda i: (0, 0)),
            pl.BlockSpec((1, H), lambda i: (0, 0)),
            pl.BlockSpec((1, H), lambda i: (0, 0)),
            pl.BlockSpec((H, wout * n_out), lambda i: (0, 0)),
        ],
        out_specs=[pl.BlockSpec((MB, wout), lambda i: (i, 0))] * n_out,
        out_shape=[_sds((NPAD, wout))] * n_out,
    )(agg, agg, agg, agg, dinv, b, ps, pq, g, be, w)


def _final_body(s0, s1, u_ref, dinv_ref, b_ref, out_ref):
    r = (s0[0] + s1[0] - u_ref[...]) * dinv_ref[...]
    out_ref[...] = r[:, :C] + b_ref[...]


def _final_call(agg3, u, dinv, b):
    return pl.pallas_call(
        _final_body,
        grid=(GR,),
        in_specs=[
            pl.BlockSpec((1, MB, 128), lambda i: (0, i, 0)),
            pl.BlockSpec((1, MB, 128), lambda i: (1, i, 0)),
            pl.BlockSpec((MB, 128), lambda i: (i, 0)),
            pl.BlockSpec((MB, 1), lambda i: (i, 0)),
            pl.BlockSpec((1, C), lambda i: (0, 0)),
        ],
        out_specs=pl.BlockSpec((MB, C), lambda i: (i, 0)),
        out_shape=_sds((N, C)),
    )(agg3, agg3, u, dinv, b)


def kernel(x, edge_idx, W1, b1, g1, be1, W2, b2, g2, be2, W3, b3):
    src = edge_idx[0].reshape(NS, ET)
    dst = edge_idx[1].reshape(NS, ET)
    spp = (NPAD - N) // NS
    pad = (N + jnp.arange(NS, dtype=jnp.int32)[:, None] * spp
           + jnp.arange(ETP - ET, dtype=jnp.int32)[None, :] % spp)
    srcp = jnp.concatenate([src, pad], axis=1).reshape(NS, NB, BB)
    dstp = jnp.concatenate([dst, pad], axis=1).reshape(NS, NB, BB)
    ones = jnp.ones((BB, 128), jnp.float32)
    zeros = jnp.zeros((NPAD, 128), jnp.float32)

    deg = _deg_call(dstp, ones, zeros)
    deg2 = deg[:, :, 0].reshape(NC, NPAD // 128, 128)
    dinv = _dinv_call(deg2).reshape(NPAD)[:N].reshape(N, 1)

    b1r, g1r, be1r = b1.reshape(1, H), g1.reshape(1, H), be1.reshape(1, H)
    b2r, g2r, be2r = b2.reshape(1, H), g2.reshape(1, H), be2.reshape(1, H)
    b3r = b3.reshape(1, C)

    u1 = _mm1_call(x, W1, dinv)
    agg1 = _agg_wide_call(*u1, srcp, dstp)
    ps1, pq1 = _stats_call(agg1, dinv, b1r)
    u2 = _bnmm_call(agg1, dinv, b1r, ps1, pq1, g1r, be1r, W2, 4, 128)
    agg2 = _agg_wide_call(*u2, srcp, dstp)
    ps2, pq2 = _stats_call(agg2, dinv, b2r)
    W3p = jnp.pad(W3, ((0, 0), (0, 128 - C)))
    (u3,) = _bnmm_call(agg2, dinv, b2r, ps2, pq2, g2r, be2r, W3p, 1, 128)
    agg3 = _agg_narrow_call(u3, srcp, dstp)
    out = _final_call(agg3, u3, dinv, b3r)
    return out


# speedup vs baseline: 2.1501x; 1.0004x over previous
import functools

import jax
import jax.numpy as jnp
from jax import lax
from jax.experimental import pallas as pl
from jax.experimental.pallas import tpu as pltpu
from jax.experimental.pallas import tpu_sc as plsc

N = 10000
E = 160000
F_IN = 256
H = 512
C = 40
EPS = 1e-5

NS = 16
NC = 2
ET = E // NS
BB = 128
NB = 80
ETP = NB * BB
NPAD = ETP
RPT = NPAD // NS
MB = 1000
GR = N // MB

_mesh = plsc.VectorSubcoreMesh(core_axis_name="c", subcore_axis_name="s")


def _sds(shape, dtype=jnp.float32):
    return jax.ShapeDtypeStruct(shape, dtype)


def _deg_body(dst_hbm, ones_hbm, zeros_hbm, deg_hbm, idx_v, ones_v, acc_sh):
    c = lax.axis_index("c")
    s = lax.axis_index("s")
    pltpu.sync_copy(dst_hbm.at[s], idx_v)
    pltpu.sync_copy(ones_hbm, ones_v)
    pltpu.sync_copy(zeros_hbm.at[pl.ds(s * RPT, RPT)],
                    acc_sh.at[pl.ds(s * RPT, RPT)])
    plsc.subcore_barrier()

    def body(j, carry):
        pltpu.sync_copy(ones_v, acc_sh.at[idx_v.at[2 * j + c]], add=True)
        return carry

    lax.fori_loop(0, NB // 2, body, 0)
    plsc.subcore_barrier()
    pltpu.sync_copy(acc_sh.at[pl.ds(s * RPT, RPT)],
                    deg_hbm.at[c, pl.ds(s * RPT, RPT)])


_deg_call = pl.kernel(
    _deg_body,
    out_type=_sds((NC, NPAD, 128)),
    mesh=_mesh,
    scratch_types=[
        pltpu.VMEM((NB, BB), jnp.int32),
        pltpu.VMEM((BB, 128), jnp.float32),
        pltpu.VMEM_SHARED((NPAD, 128), jnp.float32),
    ],
)


NBH = NB // 2


def _edge_pipeline(u_ref, srcv, dstv, acc_sh, bufs, gsem, ssem, nslots, row):
    def gstart(i, b):
        pltpu.make_async_copy(u_ref.at[srcv.at[row(i)]], bufs.at[b],
                              gsem.at[b]).start()

    def gwait(b):
        pltpu.make_async_copy(u_ref.at[srcv.at[0]], bufs.at[b],
                              gsem.at[b]).wait()

    def sstart(i, b):
        pltpu.make_async_copy(bufs.at[b], acc_sh.at[dstv.at[row(i)]],
                              ssem.at[b]).start(add=True)

    def swait(b):
        pltpu.make_async_copy(bufs.at[0], acc_sh.at[dstv.at[0]],
                              ssem.at[b]).wait()

    gstart(0, 0)

    @pl.loop(0, nslots, step=2)
    def _(j):
        for b in range(2):
            i = j + b
            gwait(b)

            @pl.when(i >= 1)
            def _():
                swait(1 - b)

            @pl.when(i + 1 < nslots)
            def _():
                gstart(i + 1, 1 - b)

            sstart(i, b)

    swait((nslots - 1) % 2)


def _agg_wide_body(u0, u1, u2, u3, src_hbm, dst_hbm, agg_hbm,
                   srcv, dstv, bufs, gsem, ssem, acc_sh):
    c = lax.axis_index("c")
    s = lax.axis_index("s")
    u_refs = (u0, u1, u2, u3)
    for chunk in range(4):
        u_ref = u_refs[chunk]

        @pl.when(c == chunk // 2)
        def _():
            pltpu.sync_copy(u_ref.at[pl.ds(s * RPT, RPT)],
                            acc_sh.at[pl.ds(s * RPT, RPT)])
            plsc.subcore_barrier()
            for p in range(2):
                pltpu.sync_copy(src_hbm.at[s, pl.ds(p * NBH, NBH)], srcv)
                pltpu.sync_copy(dst_hbm.at[s, pl.ds(p * NBH, NBH)], dstv)
                _edge_pipeline(u_ref, srcv, dstv, acc_sh, bufs, gsem, ssem,
                               NBH, lambda i: i)
            plsc.subcore_barrier()
            pltpu.sync_copy(acc_sh.at[pl.ds(s * RPT, RPT)],
                            agg_hbm.at[chunk, pl.ds(s * RPT, RPT)])
            plsc.subcore_barrier()


def _agg_scratch():
    return [
        pltpu.VMEM((NBH, BB), jnp.int32),
        pltpu.VMEM((NBH, BB), jnp.int32),
        pltpu.VMEM((2, BB, 128), jnp.float32),
        pltpu.SemaphoreType.DMA((2,)),
        pltpu.SemaphoreType.DMA((2,)),
        pltpu.VMEM_SHARED((NPAD, 128), jnp.float32),
    ]


_agg_wide_call = pl.kernel(
    _agg_wide_body,
    out_type=_sds((4, NPAD, 128)),
    mesh=_mesh,
    scratch_types=_agg_scratch(),
)


def _agg_narrow_body(u_hbm, src_hbm, dst_hbm, agg_hbm,
                     srcv, dstv, bufs, gsem, ssem, acc_sh):
    c = lax.axis_index("c")
    s = lax.axis_index("s")
    pltpu.sync_copy(u_hbm.at[pl.ds(s * RPT, RPT)],
                    acc_sh.at[pl.ds(s * RPT, RPT)])
    plsc.subcore_barrier()
    for p in range(2):
        pltpu.sync_copy(src_hbm.at[s, pl.ds(p * NBH, NBH)], srcv)
        pltpu.sync_copy(dst_hbm.at[s, pl.ds(p * NBH, NBH)], dstv)
        _edge_pipeline(u_hbm, srcv, dstv, acc_sh, bufs, gsem, ssem,
                       NBH // 2, lambda i: 2 * i + c)
    plsc.subcore_barrier()
    pltpu.sync_copy(acc_sh.at[pl.ds(s * RPT, RPT)],
                    agg_hbm.at[c, pl.ds(s * RPT, RPT)])


_agg_narrow_call = pl.kernel(
    _agg_narrow_body,
    out_type=_sds((NC, NPAD, 128)),
    mesh=_mesh,
    scratch_types=_agg_scratch(),
)


def _dinv_body(deg_ref, out_ref):
    d = deg_ref[...]
    out_ref[...] = lax.rsqrt(d[0] + d[1] + 1.0)


def _dinv_call(deg2):
    return pl.pallas_call(
        _dinv_body,
        out_shape=_sds((NPAD // 128, 128)),
    )(deg2)


def _mm1_body(x_ref, w_ref, dinv_ref, o0, o1, o2, o3):
    xs = (x_ref[...] * dinv_ref[...]).astype(jnp.bfloat16)
    u = jnp.dot(xs, w_ref[...], preferred_element_type=jnp.float32)
    for i, o in enumerate((o0, o1, o2, o3)):
        o[...] = u[:, i * 128:(i + 1) * 128]


def _mm1_call(x, w, dinv):
    return pl.pallas_call(
        _mm1_body,
        grid=(GR,),
        in_specs=[
            pl.BlockSpec((MB, F_IN), lambda i: (i, 0)),
            pl.BlockSpec((F_IN, H), lambda i: (0, 0)),
            pl.BlockSpec((MB, 1), lambda i: (i, 0)),
        ],
        out_specs=[pl.BlockSpec((MB, 128), lambda i: (i, 0))] * 4,
        out_shape=[_sds((NPAD, 128))] * 4,
    )(x, w, dinv)


def _stats_body(a0, a1, a2, a3, dinv_ref, b_ref, ps_ref, pq_ref):
    i = pl.program_id(0)
    y = jnp.concatenate([a0[0], a1[0], a2[0], a3[0]], axis=1)
    y = y * dinv_ref[...] + b_ref[...]

    @pl.when(i == 0)
    def _():
        ps_ref[...] = jnp.zeros_like(ps_ref)
        pq_ref[...] = jnp.zeros_like(pq_ref)

    ps_ref[...] += jnp.sum(y, axis=0, keepdims=True)
    pq_ref[...] += jnp.sum(y * y, axis=0, keepdims=True)


def _stats_call(agg, dinv, b):
    return pl.pallas_call(
        _stats_body,
        grid=(GR,),
        in_specs=[pl.BlockSpec((1, MB, 128), lambda i, ch=ch: (ch, i, 0))
                  for ch in range(4)]
        + [
            pl.BlockSpec((MB, 1), lambda i: (i, 0)),
            pl.BlockSpec((1, H), lambda i: (0, 0)),
        ],
        out_specs=[pl.BlockSpec((1, H), lambda i: (0, 0))] * 2,
        out_shape=[_sds((1, H))] * 2,
    )(agg, agg, agg, agg, dinv, b)


def _bnmm_body(n_out, a0, a1, a2, a3, dinv_ref, b_ref, ps_ref, pq_ref,
               g_ref, be_ref, w_ref, *outs):
    mean = ps_ref[...] / N
    var = pq_ref[...] / N - mean * mean
    scale = g_ref[...] * lax.rsqrt(var + EPS)
    shift = be_ref[...] - mean * scale
    y = jnp.concatenate([a0[0], a1[0], a2[0], a3[0]], axis=1)
    y = (y * dinv_ref[...] + b_ref[...]) * scale + shift
    z = jnp.maximum(y, 0.0)
    zs = (z * dinv_ref[...]).astype(jnp.bfloat16)
    u = jnp.dot(zs, w_ref[...], preferred_element_type=jnp.float32)
    if n_out == 1:
        outs[0][...] = u
    else:
        for i, o in enumerate(outs):
            o[...] = u[:, i * 128:(i + 1) * 128]


def _bnmm_call(agg, dinv, b, ps, pq, g, be, w, n_out, wout):
    return pl.pallas_call(
        functools.partial(_bnmm_body, n_out),
        grid=(GR,),
        in_specs=[pl.BlockSpec((1, MB, 128), lambda i, ch=ch: (ch, i, 0))
                  for ch in range(4)]
        + [
            pl.BlockSpec((MB, 1), lambda i: (i, 0)),
            pl.BlockSpec((1, H), lambda i: (0, 0)),
            pl.BlockSpec((1, H), lambda i: (0, 0)),
            pl.BlockSpec((1, H), lambda i: (0, 0)),
            pl.BlockSpec((1, H), lambda i: (0, 0)),
            pl.BlockSpec((1, H), lambda i: (0, 0)),
            pl.BlockSpec((H, wout * n_out), lambda i: (0, 0)),
        ],
        out_specs=[pl.BlockSpec((MB, wout), lambda i: (i, 0))] * n_out,
        out_shape=[_sds((NPAD, wout))] * n_out,
    )(agg, agg, agg, agg, dinv, b, ps, pq, g, be, w)


def _final_body(s0, s1, u_ref, dinv_ref, b_ref, out_ref):
    r = (s0[0] + s1[0] - u_ref[...]) * dinv_ref[...]
    out_ref[...] = r[:, :C] + b_ref[...]


def _final_call(agg3, u, dinv, b):
    return pl.pallas_call(
        _final_body,
        grid=(GR,),
        in_specs=[
            pl.BlockSpec((1, MB, 128), lambda i: (0, i, 0)),
            pl.BlockSpec((1, MB, 128), lambda i: (1, i, 0)),
            pl.BlockSpec((MB, 128), lambda i: (i, 0)),
            pl.BlockSpec((MB, 1), lambda i: (i, 0)),
            pl.BlockSpec((1, C), lambda i: (0, 0)),
        ],
        out_specs=pl.BlockSpec((MB, C), lambda i: (i, 0)),
        out_shape=_sds((N, C)),
    )(agg3, agg3, u, dinv, b)


def kernel(x, edge_idx, W1, b1, g1, be1, W2, b2, g2, be2, W3, b3):
    src = edge_idx[0].reshape(NS, ET)
    dst = edge_idx[1].reshape(NS, ET)
    spp = (NPAD - N) // NS
    pad = (N + jnp.arange(NS, dtype=jnp.int32)[:, None] * spp
           + jnp.arange(ETP - ET, dtype=jnp.int32)[None, :] % spp)
    srcp = jnp.concatenate([src, pad], axis=1).reshape(NS, NB, BB)
    dstp = jnp.concatenate([dst, pad], axis=1).reshape(NS, NB, BB)
    ones = jnp.ones((BB, 128), jnp.float32)
    zeros = jnp.zeros((NPAD, 128), jnp.float32)

    deg = _deg_call(dstp, ones, zeros)
    deg2 = deg[:, :, 0].reshape(NC, NPAD // 128, 128)
    dinv = _dinv_call(deg2).reshape(NPAD)[:N].reshape(N, 1)

    b1r, g1r, be1r = b1.reshape(1, H), g1.reshape(1, H), be1.reshape(1, H)
    b2r, g2r, be2r = b2.reshape(1, H), g2.reshape(1, H), be2.reshape(1, H)
    b3r = b3.reshape(1, C)

    u1 = _mm1_call(x, W1.astype(jnp.bfloat16), dinv)
    agg1 = _agg_wide_call(*u1, srcp, dstp)
    ps1, pq1 = _stats_call(agg1, dinv, b1r)
    u2 = _bnmm_call(agg1, dinv, b1r, ps1, pq1, g1r, be1r,
                    W2.astype(jnp.bfloat16), 4, 128)
    agg2 = _agg_wide_call(*u2, srcp, dstp)
    ps2, pq2 = _stats_call(agg2, dinv, b2r)
    W3p = jnp.pad(W3, ((0, 0), (0, 128 - C))).astype(jnp.bfloat16)
    (u3,) = _bnmm_call(agg2, dinv, b2r, ps2, pq2, g2r, be2r, W3p, 1, 128)
    agg3 = _agg_narrow_call(u3, srcp, dstp)
    out = _final_call(agg3, u3, dinv, b3r)
    return out


# speedup vs baseline: 2.1913x; 1.0192x over previous
import functools

import jax
import jax.numpy as jnp
from jax import lax
from jax.experimental import pallas as pl
from jax.experimental.pallas import tpu as pltpu
from jax.experimental.pallas import tpu_sc as plsc

N = 10000
E = 160000
F_IN = 256
H = 512
C = 40
EPS = 1e-5

NS = 16
NC = 2
ET = E // NS
BB = 128
NB = 80
ETP = NB * BB
NPAD = ETP
RPT = NPAD // NS
MB = 2000
GR = N // MB

_mesh = plsc.VectorSubcoreMesh(core_axis_name="c", subcore_axis_name="s")


def _sds(shape, dtype=jnp.float32):
    return jax.ShapeDtypeStruct(shape, dtype)


def _deg_body(dst_hbm, ones_hbm, zeros_hbm, deg_hbm, idx_v, ones_v, acc_sh):
    c = lax.axis_index("c")
    s = lax.axis_index("s")
    pltpu.sync_copy(dst_hbm.at[s], idx_v)
    pltpu.sync_copy(ones_hbm, ones_v)
    pltpu.sync_copy(zeros_hbm.at[pl.ds(s * RPT, RPT)],
                    acc_sh.at[pl.ds(s * RPT, RPT)])
    plsc.subcore_barrier()

    def body(j, carry):
        pltpu.sync_copy(ones_v, acc_sh.at[idx_v.at[2 * j + c]], add=True)
        return carry

    lax.fori_loop(0, NB // 2, body, 0)
    plsc.subcore_barrier()
    pltpu.sync_copy(acc_sh.at[pl.ds(s * RPT, RPT)],
                    deg_hbm.at[c, pl.ds(s * RPT, RPT)])


_deg_call = pl.kernel(
    _deg_body,
    out_type=_sds((NC, NPAD, 128)),
    mesh=_mesh,
    scratch_types=[
        pltpu.VMEM((NB, BB), jnp.int32),
        pltpu.VMEM((BB, 128), jnp.float32),
        pltpu.VMEM_SHARED((NPAD, 128), jnp.float32),
    ],
)


NBH = NB // 2


def _edge_pipeline(u_ref, srcv, dstv, acc_sh, bufs, gsem, ssem, nslots, row):
    def gstart(i, b):
        pltpu.make_async_copy(u_ref.at[srcv.at[row(i)]], bufs.at[b],
                              gsem.at[b]).start()

    def gwait(b):
        pltpu.make_async_copy(u_ref.at[srcv.at[0]], bufs.at[b],
                              gsem.at[b]).wait()

    def sstart(i, b):
        pltpu.make_async_copy(bufs.at[b], acc_sh.at[dstv.at[row(i)]],
                              ssem.at[b]).start(add=True)

    def swait(b):
        pltpu.make_async_copy(bufs.at[0], acc_sh.at[dstv.at[0]],
                              ssem.at[b]).wait()

    gstart(0, 0)

    @pl.loop(0, nslots, step=2)
    def _(j):
        for b in range(2):
            i = j + b
            gwait(b)

            @pl.when(i >= 1)
            def _():
                swait(1 - b)

            @pl.when(i + 1 < nslots)
            def _():
                gstart(i + 1, 1 - b)

            sstart(i, b)

    swait((nslots - 1) % 2)


def _agg_wide_body(u0, u1, u2, u3, src_hbm, dst_hbm, agg_hbm,
                   srcv, dstv, bufs, gsem, ssem, acc_sh):
    c = lax.axis_index("c")
    s = lax.axis_index("s")
    u_refs = (u0, u1, u2, u3)
    for chunk in range(4):
        u_ref = u_refs[chunk]

        @pl.when(c == chunk // 2)
        def _():
            pltpu.sync_copy(u_ref.at[pl.ds(s * RPT, RPT)],
                            acc_sh.at[pl.ds(s * RPT, RPT)])
            plsc.subcore_barrier()
            for p in range(2):
                pltpu.sync_copy(src_hbm.at[s, pl.ds(p * NBH, NBH)], srcv)
                pltpu.sync_copy(dst_hbm.at[s, pl.ds(p * NBH, NBH)], dstv)
                _edge_pipeline(u_ref, srcv, dstv, acc_sh, bufs, gsem, ssem,
                               NBH, lambda i: i)
            plsc.subcore_barrier()
            pltpu.sync_copy(acc_sh.at[pl.ds(s * RPT, RPT)],
                            agg_hbm.at[chunk, pl.ds(s * RPT, RPT)])
            plsc.subcore_barrier()


def _agg_scratch():
    return [
        pltpu.VMEM((NBH, BB), jnp.int32),
        pltpu.VMEM((NBH, BB), jnp.int32),
        pltpu.VMEM((2, BB, 128), jnp.float32),
        pltpu.SemaphoreType.DMA((2,)),
        pltpu.SemaphoreType.DMA((2,)),
        pltpu.VMEM_SHARED((NPAD, 128), jnp.float32),
    ]


_agg_wide_call = pl.kernel(
    _agg_wide_body,
    out_type=_sds((4, NPAD, 128)),
    mesh=_mesh,
    scratch_types=_agg_scratch(),
)


def _agg_narrow_body(u_hbm, src_hbm, dst_hbm, agg_hbm,
                     srcv, dstv, bufs, gsem, ssem, acc_sh):
    c = lax.axis_index("c")
    s = lax.axis_index("s")
    pltpu.sync_copy(u_hbm.at[pl.ds(s * RPT, RPT)],
                    acc_sh.at[pl.ds(s * RPT, RPT)])
    plsc.subcore_barrier()
    for p in range(2):
        pltpu.sync_copy(src_hbm.at[s, pl.ds(p * NBH, NBH)], srcv)
        pltpu.sync_copy(dst_hbm.at[s, pl.ds(p * NBH, NBH)], dstv)
        _edge_pipeline(u_hbm, srcv, dstv, acc_sh, bufs, gsem, ssem,
                       NBH // 2, lambda i: 2 * i + c)
    plsc.subcore_barrier()
    pltpu.sync_copy(acc_sh.at[pl.ds(s * RPT, RPT)],
                    agg_hbm.at[c, pl.ds(s * RPT, RPT)])


_agg_narrow_call = pl.kernel(
    _agg_narrow_body,
    out_type=_sds((NC, NPAD, 128)),
    mesh=_mesh,
    scratch_types=_agg_scratch(),
)


def _dinv_body(deg_ref, out_ref):
    d = deg_ref[...]
    out_ref[...] = lax.rsqrt(d[0] + d[1] + 1.0)


def _dinv_call(deg2):
    return pl.pallas_call(
        _dinv_body,
        out_shape=_sds((NPAD // 128, 128)),
    )(deg2)


def _mm1_body(x_ref, w_ref, dinv_ref, o0, o1, o2, o3):
    xs = x_ref[...] * dinv_ref[...]
    u = jnp.dot(xs, w_ref[...], preferred_element_type=jnp.float32)
    for i, o in enumerate((o0, o1, o2, o3)):
        o[...] = u[:, i * 128:(i + 1) * 128]


def _mm1_call(x, w, dinv):
    return pl.pallas_call(
        _mm1_body,
        grid=(GR,),
        in_specs=[
            pl.BlockSpec((MB, F_IN), lambda i: (i, 0)),
            pl.BlockSpec((F_IN, H), lambda i: (0, 0)),
            pl.BlockSpec((MB, 1), lambda i: (i, 0)),
        ],
        out_specs=[pl.BlockSpec((MB, 128), lambda i: (i, 0))] * 4,
        out_shape=[_sds((NPAD, 128))] * 4,
    )(x, w, dinv)


def _stats_body(a0, a1, a2, a3, dinv_ref, b_ref, ps_ref, pq_ref):
    i = pl.program_id(0)
    y = jnp.concatenate([a0[0], a1[0], a2[0], a3[0]], axis=1)
    y = y * dinv_ref[...] + b_ref[...]

    @pl.when(i == 0)
    def _():
        ps_ref[...] = jnp.zeros_like(ps_ref)
        pq_ref[...] = jnp.zeros_like(pq_ref)

    ps_ref[...] += jnp.sum(y, axis=0, keepdims=True)
    pq_ref[...] += jnp.sum(y * y, axis=0, keepdims=True)


def _stats_call(agg, dinv, b):
    return pl.pallas_call(
        _stats_body,
        grid=(GR,),
        in_specs=[pl.BlockSpec((1, MB, 128), lambda i, ch=ch: (ch, i, 0))
                  for ch in range(4)]
        + [
            pl.BlockSpec((MB, 1), lambda i: (i, 0)),
            pl.BlockSpec((1, H), lambda i: (0, 0)),
        ],
        out_specs=[pl.BlockSpec((1, H), lambda i: (0, 0))] * 2,
        out_shape=[_sds((1, H))] * 2,
    )(agg, agg, agg, agg, dinv, b)


def _bnmm_body(n_out, a0, a1, a2, a3, dinv_ref, b_ref, ps_ref, pq_ref,
               g_ref, be_ref, w_ref, *outs):
    mean = ps_ref[...] / N
    var = pq_ref[...] / N - mean * mean
    scale = g_ref[...] * lax.rsqrt(var + EPS)
    shift = be_ref[...] - mean * scale
    y = jnp.concatenate([a0[0], a1[0], a2[0], a3[0]], axis=1)
    y = (y * dinv_ref[...] + b_ref[...]) * scale + shift
    z = jnp.maximum(y, 0.0)
    zs = z * dinv_ref[...]
    u = jnp.dot(zs, w_ref[...], preferred_element_type=jnp.float32)
    if n_out == 1:
        outs[0][...] = u
    else:
        for i, o in enumerate(outs):
            o[...] = u[:, i * 128:(i + 1) * 128]


def _bnmm_call(agg, dinv, b, ps, pq, g, be, w, n_out, wout):
    return pl.pallas_call(
        functools.partial(_bnmm_body, n_out),
        grid=(GR,),
        in_specs=[pl.BlockSpec((1, MB, 128), lambda i, ch=ch: (ch, i, 0))
                  for ch in range(4)]
        + [
            pl.BlockSpec((MB, 1), lambda i: (i, 0)),
            pl.BlockSpec((1, H), lambda i: (0, 0)),
            pl.BlockSpec((1, H), lambda i: (0, 0)),
            pl.BlockSpec((1, H), lambda i: (0, 0)),
            pl.BlockSpec((1, H), lambda i: (0, 0)),
            pl.BlockSpec((1, H), lambda i: (0, 0)),
            pl.BlockSpec((H, wout * n_out), lambda i: (0, 0)),
        ],
        out_specs=[pl.BlockSpec((MB, wout), lambda i: (i, 0))] * n_out,
        out_shape=[_sds((NPAD, wout))] * n_out,
    )(agg, agg, agg, agg, dinv, b, ps, pq, g, be, w)


def _final_body(s0, s1, u_ref, dinv_ref, b_ref, out_ref):
    r = (s0[0] + s1[0] - u_ref[...]) * dinv_ref[...]
    out_ref[...] = r[:, :C] + b_ref[...]


def _final_call(agg3, u, dinv, b):
    return pl.pallas_call(
        _final_body,
        grid=(GR,),
        in_specs=[
            pl.BlockSpec((1, MB, 128), lambda i: (0, i, 0)),
            pl.BlockSpec((1, MB, 128), lambda i: (1, i, 0)),
            pl.BlockSpec((MB, 128), lambda i: (i, 0)),
            pl.BlockSpec((MB, 1), lambda i: (i, 0)),
            pl.BlockSpec((1, C), lambda i: (0, 0)),
        ],
        out_specs=pl.BlockSpec((MB, C), lambda i: (i, 0)),
        out_shape=_sds((N, C)),
    )(agg3, agg3, u, dinv, b)


def kernel(x, edge_idx, W1, b1, g1, be1, W2, b2, g2, be2, W3, b3):
    src = edge_idx[0].reshape(NS, ET)
    dst = edge_idx[1].reshape(NS, ET)
    spp = (NPAD - N) // NS
    pad = (N + jnp.arange(NS, dtype=jnp.int32)[:, None] * spp
           + jnp.arange(ETP - ET, dtype=jnp.int32)[None, :] % spp)
    srcp = jnp.concatenate([src, pad], axis=1).reshape(NS, NB, BB)
    dstp = jnp.concatenate([dst, pad], axis=1).reshape(NS, NB, BB)
    ones = jnp.ones((BB, 128), jnp.float32)
    zeros = jnp.zeros((NPAD, 128), jnp.float32)

    deg = _deg_call(dstp, ones, zeros)
    deg2 = deg[:, :, 0].reshape(NC, NPAD // 128, 128)
    dinv = _dinv_call(deg2).reshape(NPAD)[:N].reshape(N, 1)

    b1r, g1r, be1r = b1.reshape(1, H), g1.reshape(1, H), be1.reshape(1, H)
    b2r, g2r, be2r = b2.reshape(1, H), g2.reshape(1, H), be2.reshape(1, H)
    b3r = b3.reshape(1, C)

    u1 = _mm1_call(x, W1, dinv)
    agg1 = _agg_wide_call(*u1, srcp, dstp)
    ps1, pq1 = _stats_call(agg1, dinv, b1r)
    u2 = _bnmm_call(agg1, dinv, b1r, ps1, pq1, g1r, be1r, W2, 4, 128)
    agg2 = _agg_wide_call(*u2, srcp, dstp)
    ps2, pq2 = _stats_call(agg2, dinv, b2r)
    W3p = jnp.pad(W3, ((0, 0), (0, 128 - C)))
    (u3,) = _bnmm_call(agg2, dinv, b2r, ps2, pq2, g2r, be2r, W3p, 1, 128)
    agg3 = _agg_narrow_call(u3, srcp, dstp)
    out = _final_call(agg3, u3, dinv, b3r)
    return out
